# baseline (speedup 1.0000x reference)
import functools

import jax
import jax.numpy as jnp
from jax import lax
from jax.experimental import pallas as pl
from jax.experimental.pallas import tpu as pltpu
from jax.experimental.pallas import tpu_sc as plsc


@functools.lru_cache(maxsize=None)
def _build_sc(B, L, T, V, E):
    NW = 32
    NB = B // NW
    TOK = L * T
    TOKP = 208
    NA, NBI = 112, 96
    GROWS = L + 2
    RMAX = 3 * (V - 2)
    OUTROW = L * E
    BLK = 3 * E

    mesh = plsc.VectorSubcoreMesh(core_axis_name="c", subcore_axis_name="s")

    idx_types = [pltpu.VMEM((NA,), jnp.int32), pltpu.VMEM((NBI,), jnp.int32)]

    @functools.partial(
        pl.kernel,
        mesh=mesh,
        compiler_params=pltpu.CompilerParams(
            use_tc_tiling_on_sc=False, needs_layout_passes=False),
        out_type=jax.ShapeDtypeStruct((B * OUTROW,), jnp.float32),
        scratch_types=[
            pltpu.VMEM((NB * TOK + 16,), jnp.int32),
        ] + idx_types * 6 + [
            pltpu.VMEM((TOKP, E), jnp.float32),
            pltpu.VMEM((TOKP, E), jnp.float32),
            pltpu.VMEM((TOKP, E), jnp.float32),
            pltpu.VMEM((TOKP, E), jnp.float32),
            pltpu.VMEM((TOKP, E), jnp.float32),
            pltpu.VMEM((TOKP, E), jnp.float32),
            pltpu.VMEM((GROWS * BLK,), jnp.float32),
            pltpu.VMEM((OUTROW,), jnp.float32),
            pltpu.VMEM((2, 128), jnp.float32),
            pltpu.VMEM((64,), jnp.float32),
            pltpu.VMEM((64,), jnp.float32),
            pltpu.VMEM((3, E), jnp.float32),
            pltpu.SemaphoreType.DMA,
            pltpu.SemaphoreType.DMA,
        ],
    )
    def encode(seq_hbm, w_hbm, out_hbm, seq_v,
               ia00, ib00, ia01, ib01, ia02, ib02,
               ia10, ib10, ia11, ib11, ia12, ib12,
               e00, e01, e02, e10, e11, e12,
               g_v, out_v, nz_v, den_v, rcp_v, w0_v, sem0, sem1):
        idx = ((( ia00, ib00), (ia01, ib01), (ia02, ib02)),
               (((ia10), ib10), (ia11, ib11), (ia12, ib12)))
        e_v = ((e00, e01, e02), (e10, e11, e12))
        sems = (sem0, sem1)

        wid = lax.axis_index("s") * 2 + lax.axis_index("c")
        b0 = wid * NB
        zero16f = jnp.zeros((16,), jnp.float32)
        zero16i = jnp.zeros((16,), jnp.int32)

        pltpu.sync_copy(seq_hbm.at[pl.ds(b0 * TOK, NB * TOK)],
                        seq_v.at[pl.ds(0, NB * TOK)])
        seq_v[pl.ds(NB * TOK, 16)] = zero16i
        pltpu.sync_copy(w_hbm.at[pl.ds(0, 3)], w0_v)
        w0_vals = [[w0_v[p, pl.ds(16 * c, 16)] for c in range(E // 16)]
                   for p in range(3)]
        iota16 = lax.iota(jnp.int32, 16)
        for j in range(BLK // 16):
            g_v[pl.ds(16 * j, 16)] = zero16f
            g_v[pl.ds((GROWS - 1) * BLK + 16 * j, 16)] = zero16f
        for k in range(TOK // 16, TOKP // 16 + 3):
            nz_v[(16 * k) // 128, pl.ds((16 * k) % 128, 16)] = zero16f

        def fill_indices(r, buf):
            base = r * TOK
            for k in range(TOKP // 16):
                v = seq_v[pl.ds(base + 16 * k, 16)]
                p0 = jnp.clip(v * 3 - 3, 0, RMAX)
                for p in range(3):
                    if 16 * k < NA:
                        idx[buf][p][0][pl.ds(16 * k, 16)] = p0 + p
                    else:
                        idx[buf][p][1][pl.ds(16 * k - NA, 16)] = p0 + p
                if 16 * k < TOK:
                    nz_v[(16 * k) // 128, pl.ds((16 * k) % 128, 16)] = (
                        jnp.where(v != 0, 1.0, 0.0))

        def copies(buf):
            out = []
            for p in range(3):
                out.append(pltpu.make_async_copy(
                    w_hbm.at[idx[buf][p][0]],
                    e_v[buf][p].at[pl.ds(0, NA)], sems[buf]))
                out.append(pltpu.make_async_copy(
                    w_hbm.at[idx[buf][p][1]],
                    e_v[buf][p].at[pl.ds(NA, NBI)], sems[buf]))
            return out

        def fire(buf):
            for c in copies(buf):
                c.start()

        def drain(buf):
            for c in copies(buf):
                c.wait()

        fill_indices(0, 0)
        fire(0)

        def process(r, buf):
            for g in range((L + 15) // 16):
                lvec = (iota16 + 16 * g) * T

                def nzg(tok):
                    return plsc.load_gather(
                        nz_v, [lax.shift_right_logical(tok, 7),
                               lax.bitwise_and(tok, 127)])

                s = nzg(lvec)
                for t in range(1, T):
                    s = s + nzg(lvec + t)
                den_v[pl.ds(16 * g, 16)] = s
                rcp_v[pl.ds(16 * g, 16)] = jnp.where(
                    s > 0.0, 1.0 / jnp.maximum(s, 1.0), 0.0)

            drain(buf)

            @pl.when(r + 1 < NB)
            def _():
                fill_indices(r + 1, 1 - buf)
                fire(1 - buf)

            ev = e_v[buf]

            @plsc.parallel_loop(0, L, step=1, unroll=2)
            def acc_body(l):
                lsplat = jnp.full((16,), l, jnp.int32)
                c0b = 4.0 - plsc.load_gather(den_v, [lsplat])
                for p in range(3):
                    for c in range(E // 16):
                        sl = pl.ds(16 * c, 16)
                        a = ev[p][T * l, sl] + ev[p][T * l + 1, sl]
                        a = a + ev[p][T * l + 2, sl] + ev[p][T * l + 3, sl]
                        a = a - c0b * w0_vals[p][c]
                        g_v[pl.ds((l + 1) * BLK + p * E + 16 * c, 16)] = a

            @plsc.parallel_loop(0, L, step=1, unroll=2)
            def out_body(l):
                rb = plsc.load_gather(rcp_v, [jnp.full((16,), l, jnp.int32)])
                for c in range(E // 16):
                    x = g_v[pl.ds(l * BLK + 16 * c, 16)]
                    x = x + g_v[pl.ds((l + 1) * BLK + E + 16 * c, 16)]
                    x = x + g_v[pl.ds((l + 2) * BLK + 2 * E + 16 * c, 16)]
                    x = x * rb
                    e = jnp.exp(x + x)
                    out_v[pl.ds(l * E + 16 * c, 16)] = 1.0 - 2.0 / (e + 1.0)
            pltpu.sync_copy(out_v, out_hbm.at[pl.ds((b0 + r) * OUTROW, OUTROW)])

        def row_pair(rr, _):
            process(2 * rr, 0)
            process(2 * rr + 1, 1)
            return 0

        lax.fori_loop(0, NB // 2, row_pair, 0)

    return encode


def kernel(seq, W):
    B, L, T = seq.shape
    E = W.shape[1]
    V = W.shape[0] // 3 + 1
    out = _build_sc(B, L, T, V, E)(seq.reshape(-1), W)
    return out.reshape(B, L, E)


# speedup vs baseline: 1.0344x; 1.0344x over previous
import functools

import jax
import jax.numpy as jnp
from jax import lax
from jax.experimental import pallas as pl
from jax.experimental.pallas import tpu as pltpu
from jax.experimental.pallas import tpu_sc as plsc


@functools.lru_cache(maxsize=None)
def _build_sc(B, L, T, V, E):
    NW = 32
    NB = B // NW
    TOK = L * T
    TOKP = 208
    NA, NBI = 112, 96
    GROWS = L + 2
    RMAX = 3 * (V - 2)
    OUTROW = L * E
    BLK = 3 * E

    mesh = plsc.VectorSubcoreMesh(core_axis_name="c", subcore_axis_name="s")

    idx_types = [pltpu.VMEM((NA,), jnp.int32), pltpu.VMEM((NBI,), jnp.int32)]

    @functools.partial(
        pl.kernel,
        mesh=mesh,
        compiler_params=pltpu.CompilerParams(
            use_tc_tiling_on_sc=False, needs_layout_passes=False),
        out_type=jax.ShapeDtypeStruct((B * OUTROW,), jnp.float32),
        scratch_types=[
            pltpu.VMEM((NB * TOK + 16,), jnp.int32),
        ] + idx_types * 6 + [
            pltpu.VMEM((TOKP, E), jnp.float32),
            pltpu.VMEM((TOKP, E), jnp.float32),
            pltpu.VMEM((TOKP, E), jnp.float32),
            pltpu.VMEM((TOKP, E), jnp.float32),
            pltpu.VMEM((TOKP, E), jnp.float32),
            pltpu.VMEM((TOKP, E), jnp.float32),
            pltpu.VMEM((OUTROW,), jnp.float32),
            pltpu.VMEM((2, 128), jnp.float32),
            pltpu.VMEM((64,), jnp.float32),
            pltpu.VMEM((64,), jnp.float32),
            pltpu.VMEM((3, E), jnp.float32),
            pltpu.SemaphoreType.DMA,
            pltpu.SemaphoreType.DMA,
        ],
    )
    def encode(seq_hbm, w_hbm, out_hbm, seq_v,
               ia00, ib00, ia01, ib01, ia02, ib02,
               ia10, ib10, ia11, ib11, ia12, ib12,
               e00, e01, e02, e10, e11, e12,
               out_v, nz_v, den_v, rcp_v, w0_v, sem0, sem1):
        idx = ((( ia00, ib00), (ia01, ib01), (ia02, ib02)),
               (((ia10), ib10), (ia11, ib11), (ia12, ib12)))
        e_v = ((e00, e01, e02), (e10, e11, e12))
        sems = (sem0, sem1)

        wid = lax.axis_index("s") * 2 + lax.axis_index("c")
        b0 = wid * NB
        zero16f = jnp.zeros((16,), jnp.float32)
        zero16i = jnp.zeros((16,), jnp.int32)

        pltpu.sync_copy(seq_hbm.at[pl.ds(b0 * TOK, NB * TOK)],
                        seq_v.at[pl.ds(0, NB * TOK)])
        seq_v[pl.ds(NB * TOK, 16)] = zero16i
        pltpu.sync_copy(w_hbm.at[pl.ds(0, 3)], w0_v)
        w0_vals = [[w0_v[p, pl.ds(16 * c, 16)] for c in range(E // 16)]
                   for p in range(3)]
        iota16 = lax.iota(jnp.int32, 16)
        for k in range(TOK // 16, TOKP // 16 + 3):
            nz_v[(16 * k) // 128, pl.ds((16 * k) % 128, 16)] = zero16f

        def fill_indices(r, buf):
            base = r * TOK
            for k in range(TOKP // 16):
                v = seq_v[pl.ds(base + 16 * k, 16)]
                p0 = jnp.clip(v * 3 - 3, 0, RMAX)
                for p in range(3):
                    if 16 * k < NA:
                        idx[buf][p][0][pl.ds(16 * k, 16)] = p0 + p
                    else:
                        idx[buf][p][1][pl.ds(16 * k - NA, 16)] = p0 + p
                if 16 * k < TOK:
                    nz_v[(16 * k) // 128, pl.ds((16 * k) % 128, 16)] = (
                        jnp.where(v != 0, 1.0, 0.0))

        def copies(buf):
            out = []
            for p in range(3):
                out.append(pltpu.make_async_copy(
                    w_hbm.at[idx[buf][p][0]],
                    e_v[buf][p].at[pl.ds(0, NA)], sems[buf]))
                out.append(pltpu.make_async_copy(
                    w_hbm.at[idx[buf][p][1]],
                    e_v[buf][p].at[pl.ds(NA, NBI)], sems[buf]))
            return out

        def fire(buf):
            for c in copies(buf):
                c.start()

        def drain(buf):
            for c in copies(buf):
                c.wait()

        fill_indices(0, 0)
        fire(0)

        def process(r, buf):
            for g in range((L + 15) // 16):
                lvec = (iota16 + 16 * g) * T

                def nzg(tok):
                    return plsc.load_gather(
                        nz_v, [lax.shift_right_logical(tok, 7),
                               lax.bitwise_and(tok, 127)])

                s = nzg(lvec)
                for t in range(1, T):
                    s = s + nzg(lvec + t)
                den_v[pl.ds(16 * g, 16)] = s
                rcp_v[pl.ds(16 * g, 16)] = jnp.where(
                    s > 0.0, 1.0 / jnp.maximum(s, 1.0), 0.0)

            drain(buf)

            @pl.when(r + 1 < NB)
            def _():
                fill_indices(r + 1, 1 - buf)
                fire(1 - buf)

            ev = e_v[buf]

            NC4 = E // 16

            def accum(l):
                c0b = 4.0 - plsc.load_gather(
                    den_v, [jnp.full((16,), l, jnp.int32)])
                parts = []
                for p in range(3):
                    row = []
                    for c in range(NC4):
                        sl = pl.ds(16 * c, 16)
                        a = ev[p][T * l, sl] + ev[p][T * l + 1, sl]
                        a = a + ev[p][T * l + 2, sl] + ev[p][T * l + 3, sl]
                        row.append(a - c0b * w0_vals[p][c])
                    parts.append(tuple(row))
                return tuple(parts)

            def emit(j, x):
                rb = plsc.load_gather(rcp_v, [jnp.full((16,), j, jnp.int32)])
                for c in range(NC4):
                    y = x[c] * rb
                    e = jnp.exp(y + y)
                    out_v[pl.ds(j * E + 16 * c, 16)] = 1.0 - 2.0 / (e + 1.0)

            p0, q0, _r0 = accum(0)

            @plsc.parallel_loop(1, L, step=1, unroll=2, carry=(q0, p0))
            def fused(l, carry):
                s, u = carry
                p, q, rr2 = accum(l)
                emit(l - 1, tuple(s[c] + rr2[c] for c in range(NC4)))
                return (tuple(u[c] + q[c] for c in range(NC4)), p)

            s_fin, _u_fin = fused
            emit(L - 1, s_fin)
            pltpu.sync_copy(out_v, out_hbm.at[pl.ds((b0 + r) * OUTROW, OUTROW)])

        def row_pair(rr, _):
            process(2 * rr, 0)
            process(2 * rr + 1, 1)
            return 0

        lax.fori_loop(0, NB // 2, row_pair, 0)

    return encode


def kernel(seq, W):
    B, L, T = seq.shape
    E = W.shape[1]
    V = W.shape[0] // 3 + 1
    out = _build_sc(B, L, T, V, E)(seq.reshape(-1), W)
    return out.reshape(B, L, E)
